# baseline (speedup 1.0000x reference)
import functools

import jax
import jax.numpy as jnp
from jax import lax
from jax.experimental import pallas as pl
from jax.experimental.pallas import tpu as pltpu
from jax.experimental.pallas import tpu_sc as plsc

_NUM_BUCKETS = 2048
_D = 128
_DM = 1024
_T = 8192
_B = 4
_N = _B * _T

_NC = 2
_NS = 16
_NW = _NC * _NS
_LANES = 16

_PW = _N // _NW
_RPW = _T // _PW
_C = 128
_NCHUNK = _PW // _C


def _make_sc_gather():
    mesh = plsc.VectorSubcoreMesh(
        core_axis_name="c", subcore_axis_name="s",
        num_cores=_NC, num_subcores=_NS)

    @functools.partial(
        pl.kernel,
        out_type=jax.ShapeDtypeStruct((_N, _D // 2), jnp.int32),
        mesh=mesh,
        scratch_types=[
            pltpu.VMEM((_PW,), jnp.int32),
            pltpu.VMEM((_PW + 8,), jnp.int32),
            pltpu.VMEM((_NCHUNK, _C), jnp.int32),
            pltpu.VMEM((4, _C, _D // 2), jnp.int32),
            pltpu.SemaphoreType.DMA,
            pltpu.SemaphoreType.DMA,
        ],
        compiler_params=pltpu.CompilerParams(use_tc_tiling_on_sc=False),
    )
    def sc_gather(ids_hbm, table_hbm, out_hbm, idsv, prevv, hashv, bufs,
                  gsem, wsem):
        wid = lax.axis_index("s") * _NC + lax.axis_index("c")
        base = wid * _PW

        pltpu.sync_copy(ids_hbm.at[pl.ds(base, _PW)], idsv)

        @pl.when(wid == 0)
        def _():
            pltpu.sync_copy(ids_hbm.at[pl.ds(0, _PW)], prevv.at[pl.ds(8, _PW)])

        @pl.when(wid != 0)
        def _():
            pltpu.sync_copy(ids_hbm.at[pl.ds(base - 8, _PW + 8)], prevv)

        lane = lax.iota(jnp.int32, _LANES)
        not_row_start = jnp.minimum(wid % _RPW, 1)
        factor0 = jnp.maximum(jnp.minimum(lane, 1), not_row_start)
        for g in range(_PW // _LANES):
            cur = idsv[pl.ds(g * _LANES, _LANES)]
            prev = prevv[pl.ds(g * _LANES + 7, _LANES)]
            if g == 0:
                prev = prev * factor0
            h = (prev * 1056 + cur) & (_NUM_BUCKETS - 1)
            hashv[g // (_C // _LANES),
                  pl.ds((g % (_C // _LANES)) * _LANES, _LANES)] = h

        gathers = {}
        writes = {}
        for c in range(_NCHUNK):
            if c - 4 in writes:
                writes.pop(c - 4).wait()
            gathers[c] = pltpu.async_copy(
                table_hbm.at[hashv.at[c]], bufs.at[c % 4], gsem)
            if c - 1 in gathers:
                gathers.pop(c - 1).wait()
                writes[c - 1] = pltpu.async_copy(
                    bufs.at[(c - 1) % 4],
                    out_hbm.at[pl.ds(base + (c - 1) * _C, _C)], wsem)
        c = _NCHUNK - 1
        gathers.pop(c).wait()
        writes[c] = pltpu.async_copy(
            bufs.at[c % 4], out_hbm.at[pl.ds(base + c * _C, _C)], wsem)
        for w in writes.values():
            w.wait()

    return sc_gather


_sc_gather = _make_sc_gather()


def _mm_body(a_ref, w_ref, o_ref):
    a2 = pltpu.bitcast(a_ref[...], jnp.bfloat16)
    a3 = a2.reshape(a_ref.shape[0], 2, _D // 2)
    a = jnp.concatenate([a3[:, 0, :], a3[:, 1, :]], axis=1)
    o_ref[...] = jnp.dot(a, w_ref[...].astype(jnp.bfloat16),
                         preferred_element_type=jnp.float32)


_BLK = 4096


@jax.jit
def kernel(input_ids, bigram_embed, proj_w):
    b, t = input_ids.shape
    tb = bigram_embed.astype(jnp.bfloat16)
    table_packed = jax.lax.bitcast_convert_type(
        jnp.stack([tb[:, :_D // 2], tb[:, _D // 2:]], axis=-1), jnp.int32)
    gathered = _sc_gather(input_ids.reshape(-1).astype(jnp.int32),
                          table_packed)
    out = pl.pallas_call(
        _mm_body,
        grid=(_N // _BLK,),
        in_specs=[
            pl.BlockSpec((_BLK, _D // 2), lambda i: (i, 0)),
            pl.BlockSpec((_D, _DM), lambda i: (0, 0)),
        ],
        out_specs=pl.BlockSpec((_BLK, _DM), lambda i: (i, 0)),
        out_shape=jax.ShapeDtypeStruct((_N, _DM), jnp.float32),
    )(gathered, proj_w)
    return out.reshape(b, t, _DM)


# speedup vs baseline: 1.2321x; 1.2321x over previous
import functools

import jax
import jax.numpy as jnp
from jax import lax
from jax.experimental import pallas as pl
from jax.experimental.pallas import tpu as pltpu
from jax.experimental.pallas import tpu_sc as plsc

_NUM_BUCKETS = 2048
_D = 128
_DM = 1024
_T = 8192
_B = 4
_N = _B * _T

_NC = 2
_NS = 16
_NW = _NC * _NS
_LANES = 16

_PW = _N // _NW
_RPW = _T // _PW
_C = 128
_NCHUNK = _PW // _C


def _make_sc_gather():
    mesh = plsc.VectorSubcoreMesh(
        core_axis_name="c", subcore_axis_name="s",
        num_cores=_NC, num_subcores=_NS)

    @functools.partial(
        pl.kernel,
        out_type=jax.ShapeDtypeStruct((_N, _D), jnp.float32),
        mesh=mesh,
        scratch_types=[
            pltpu.VMEM((_PW,), jnp.int32),
            pltpu.VMEM((_PW + 8,), jnp.int32),
            pltpu.VMEM((_NCHUNK, _C), jnp.int32),
            pltpu.VMEM((4, _C, _D), jnp.float32),
            pltpu.SemaphoreType.DMA,
            pltpu.SemaphoreType.DMA,
        ],
    )
    def sc_gather(ids_hbm, table_hbm, out_hbm, idsv, prevv, hashv, bufs,
                  gsem, wsem):
        wid = lax.axis_index("s") * _NC + lax.axis_index("c")
        base = wid * _PW

        ids_cp = pltpu.async_copy(ids_hbm.at[pl.ds(base, _PW)], idsv, gsem)

        @pl.when(wid == 0)
        def _():
            pltpu.sync_copy(ids_hbm.at[pl.ds(0, _PW)], prevv.at[pl.ds(8, _PW)])

        @pl.when(wid != 0)
        def _():
            pltpu.sync_copy(ids_hbm.at[pl.ds(base - 8, _PW + 8)], prevv)

        ids_cp.wait()

        lane = lax.iota(jnp.int32, _LANES)
        not_row_start = jnp.minimum(wid % _RPW, 1)
        factor0 = jnp.maximum(jnp.minimum(lane, 1), not_row_start)
        for g in range(_PW // _LANES):
            cur = idsv[pl.ds(g * _LANES, _LANES)]
            prev = prevv[pl.ds(g * _LANES + 7, _LANES)]
            if g == 0:
                prev = prev * factor0
            h = (prev * 1056 + cur) & (_NUM_BUCKETS - 1)
            hashv[g // (_C // _LANES),
                  pl.ds((g % (_C // _LANES)) * _LANES, _LANES)] = h

        gathers = {}
        writes = {}
        for c in range(_NCHUNK):
            if c - 4 in writes:
                writes.pop(c - 4).wait()
            gathers[c] = pltpu.async_copy(
                table_hbm.at[hashv.at[c]], bufs.at[c % 4], gsem)
            if c - 1 in gathers:
                gathers.pop(c - 1).wait()
                writes[c - 1] = pltpu.async_copy(
                    bufs.at[(c - 1) % 4],
                    out_hbm.at[pl.ds(base + (c - 1) * _C, _C)], wsem)
        c = _NCHUNK - 1
        gathers.pop(c).wait()
        writes[c] = pltpu.async_copy(
            bufs.at[c % 4], out_hbm.at[pl.ds(base + c * _C, _C)], wsem)
        for w in writes.values():
            w.wait()

    return sc_gather


_sc_gather = _make_sc_gather()


def _mm_body(a_ref, w_ref, o_ref):
    o_ref[...] = jnp.dot(a_ref[...].astype(jnp.bfloat16),
                         w_ref[...].astype(jnp.bfloat16),
                         preferred_element_type=jnp.float32)


_BLK = 4096


@jax.jit
def kernel(input_ids, bigram_embed, proj_w):
    b, t = input_ids.shape
    gathered = _sc_gather(input_ids.reshape(-1).astype(jnp.int32),
                          bigram_embed)
    out = pl.pallas_call(
        _mm_body,
        grid=(_N // _BLK,),
        in_specs=[
            pl.BlockSpec((_BLK, _D), lambda i: (i, 0)),
            pl.BlockSpec((_D, _DM), lambda i: (0, 0)),
        ],
        out_specs=pl.BlockSpec((_BLK, _DM), lambda i: (i, 0)),
        out_shape=jax.ShapeDtypeStruct((_N, _DM), jnp.float32),
    )(gathered, proj_w)
    return out.reshape(b, t, _DM)


# speedup vs baseline: 1.2329x; 1.0007x over previous
import functools

import jax
import jax.numpy as jnp
from jax import lax
from jax.experimental import pallas as pl
from jax.experimental.pallas import tpu as pltpu
from jax.experimental.pallas import tpu_sc as plsc

_NUM_BUCKETS = 2048
_D = 128
_DM = 1024
_T = 8192
_B = 4
_N = _B * _T

_NC = 2
_NS = 16
_NW = _NC * _NS
_LANES = 16

_PW = _N // _NW
_RPW = _T // _PW
_C = 128
_NCHUNK = _PW // _C


def _make_sc_gather():
    mesh = plsc.VectorSubcoreMesh(
        core_axis_name="c", subcore_axis_name="s",
        num_cores=_NC, num_subcores=_NS)

    @functools.partial(
        pl.kernel,
        out_type=jax.ShapeDtypeStruct((_N, _D), jnp.float32),
        mesh=mesh,
        scratch_types=[
            pltpu.VMEM((_PW,), jnp.int32),
            pltpu.VMEM((_PW + 8,), jnp.int32),
            pltpu.VMEM((_NCHUNK, _C), jnp.int32),
            pltpu.VMEM((4, _C, _D), jnp.float32),
            pltpu.SemaphoreType.DMA,
            pltpu.SemaphoreType.DMA,
        ],
    )
    def sc_gather(ids_hbm, table_hbm, out_hbm, idsv, prevv, hashv, bufs,
                  gsem, wsem):
        wid = lax.axis_index("s") * _NC + lax.axis_index("c")
        base = wid * _PW

        ids_cp = pltpu.async_copy(ids_hbm.at[pl.ds(base, _PW)], idsv, gsem)

        @pl.when(wid == 0)
        def _():
            pltpu.sync_copy(ids_hbm.at[pl.ds(0, _PW)], prevv.at[pl.ds(8, _PW)])

        @pl.when(wid != 0)
        def _():
            pltpu.sync_copy(ids_hbm.at[pl.ds(base - 8, _PW + 8)], prevv)

        ids_cp.wait()

        lane = lax.iota(jnp.int32, _LANES)
        not_row_start = jnp.minimum(wid % _RPW, 1)
        factor0 = jnp.maximum(jnp.minimum(lane, 1), not_row_start)
        for g in range(_PW // _LANES):
            cur = idsv[pl.ds(g * _LANES, _LANES)]
            prev = prevv[pl.ds(g * _LANES + 7, _LANES)]
            if g == 0:
                prev = prev * factor0
            h = (prev * 1056 + cur) & (_NUM_BUCKETS - 1)
            hashv[g // (_C // _LANES),
                  pl.ds((g % (_C // _LANES)) * _LANES, _LANES)] = h

        gathers = {}
        writes = {}
        for c in range(_NCHUNK):
            if c - 4 in writes:
                writes.pop(c - 4).wait()
            gathers[c] = pltpu.async_copy(
                table_hbm.at[hashv.at[c]], bufs.at[c % 4], gsem)
            if c - 2 in gathers:
                gathers.pop(c - 2).wait()
                writes[c - 2] = pltpu.async_copy(
                    bufs.at[(c - 2) % 4],
                    out_hbm.at[pl.ds(base + (c - 2) * _C, _C)], wsem)
        for c in sorted(gathers):
            gathers.pop(c).wait()
            writes[c] = pltpu.async_copy(
                bufs.at[c % 4], out_hbm.at[pl.ds(base + c * _C, _C)], wsem)
        for w in writes.values():
            w.wait()

    return sc_gather


_sc_gather = _make_sc_gather()


def _mm_body(a_ref, w_ref, o_ref):
    o_ref[...] = jnp.dot(a_ref[...].astype(jnp.bfloat16),
                         w_ref[...].astype(jnp.bfloat16),
                         preferred_element_type=jnp.float32)


_BLK = 4096


@jax.jit
def kernel(input_ids, bigram_embed, proj_w):
    b, t = input_ids.shape
    gathered = _sc_gather(input_ids.reshape(-1).astype(jnp.int32),
                          bigram_embed)
    out = pl.pallas_call(
        _mm_body,
        grid=(_N // _BLK,),
        in_specs=[
            pl.BlockSpec((_BLK, _D), lambda i: (i, 0)),
            pl.BlockSpec((_D, _DM), lambda i: (0, 0)),
        ],
        out_specs=pl.BlockSpec((_BLK, _DM), lambda i: (i, 0)),
        out_shape=jax.ShapeDtypeStruct((_N, _DM), jnp.float32),
    )(gathered, proj_w)
    return out.reshape(b, t, _DM)
